# Initial kernel scaffold; baseline (speedup 1.0000x reference)
import jax
import jax.numpy as jnp
from jax.experimental import pallas as pl


def kernel(x, emb_table, w1, b1, w2, b2):
    raise NotImplementedError("write your pallas kernel here")


# speedup vs baseline: 57.9718x; 57.9718x over previous
import jax
import jax.numpy as jnp
from jax import lax
from jax.experimental import pallas as pl
from jax.experimental.pallas import tpu as pltpu
from jax.experimental.pallas import tpu_sc as plsc

NC, NS, L = 2, 16, 16
NW = NC * NS
B, H, D = 16384, 200, 10
DP = 16
RPW = B // NW
CH = 16
G = RPW // CH
IDX_T = 128
NT = CH * H // IDX_T
NHID = 5
NUM_ROWS = 1000008
WB1, WW2, WB2 = NHID * L, NHID * L + NHID, NHID * L + 2 * NHID


def _body(x_hbm, tbl_hbm, wts_hbm, out_hbm,
          idx_v, rows_v, wts_v, out_v, sem):
    wid = lax.axis_index("s") * NC + lax.axis_index("c")
    tbl_r = tbl_hbm

    pltpu.sync_copy(wts_hbm, wts_v)
    w1vecs = [wts_v[pl.ds(k * L, L)] for k in range(NHID)]
    w1s = [[w1vecs[k][d] for d in range(D)] for k in range(NHID)]
    tail = wts_v[pl.ds(WB1, L)]
    b1s = [tail[k] for k in range(NHID)]
    w2s = [tail[NHID + k] for k in range(NHID)]
    b2s = tail[2 * NHID]

    lane = lax.iota(jnp.int32, L)

    def chunk(g, carry):
        pltpu.sync_copy(x_hbm.at[wid, g], idx_v)
        cps = [pltpu.async_copy(tbl_r.at[idx_v.at[j]],
                                rows_v.at[pl.ds(j * IDX_T, IDX_T)], sem)
               for j in range(NT)]
        for cp in cps:
            cp.wait()

        zvec = jnp.zeros((L,), jnp.float32)
        for s in range(CH):
            base = s * H

            def inner(u, accs):
                a0, a1, a2, a3 = accs
                r = base + u * 8
                a0 = a0 + rows_v[r]
                a1 = a1 + rows_v[r + 1]
                a2 = a2 + rows_v[r + 2]
                a3 = a3 + rows_v[r + 3]
                a0 = a0 + rows_v[r + 4]
                a1 = a1 + rows_v[r + 5]
                a2 = a2 + rows_v[r + 6]
                a3 = a3 + rows_v[r + 7]
                return a0, a1, a2, a3

            z16 = jnp.zeros((L,), jnp.float32)
            a0, a1, a2, a3 = lax.fori_loop(0, H // 8, inner,
                                           (z16, z16, z16, z16))
            acc = (a0 + a1) + (a2 + a3)

            es = [acc[d] for d in range(D)]
            z = b2s
            for k in range(NHID):
                t = b1s[k]
                for d in range(D):
                    t = t + es[d] * w1s[k][d]
                z = z + w2s[k] * jnp.maximum(t, 0.0)
            zvec = jnp.where(lane == s, z, zvec)

        out_v[pl.ds(g * CH, CH)] = 1.0 / (1.0 + jnp.exp(-zvec))
        return carry

    lax.fori_loop(0, G, chunk, 0)
    pltpu.sync_copy(out_v, out_hbm.at[pl.ds(wid * RPW, RPW)])


def kernel(x, emb_table, w1, b1, w2, b2):
    tbl = jnp.pad(emb_table,
                  ((0, NUM_ROWS - emb_table.shape[0]), (0, DP - D)))
    x_r = x.reshape(NW, G, NT, IDX_T)
    w1p = jnp.pad((w1 / H).astype(jnp.float32), ((0, 0), (0, L - D)))
    wts = jnp.concatenate([
        w1p.reshape(-1),
        b1.reshape(-1),
        w2.reshape(-1),
        b2.reshape(-1),
        jnp.zeros((5,), jnp.float32),
    ])

    run = pl.kernel(
        _body,
        out_type=jax.ShapeDtypeStruct((B,), jnp.float32),
        mesh=plsc.VectorSubcoreMesh(core_axis_name="c", subcore_axis_name="s"),
        compiler_params=pltpu.CompilerParams(use_tc_tiling_on_sc=False),
        scratch_types=[
            pltpu.VMEM((NT, IDX_T), jnp.int32),
            pltpu.VMEM((CH * H, DP), jnp.float32),
            pltpu.VMEM((96,), jnp.float32),
            pltpu.VMEM((RPW,), jnp.float32),
            pltpu.SemaphoreType.DMA,
        ],
    )
    out = run(x_r, tbl, wts)
    return out.reshape(B, 1)


# speedup vs baseline: 64.8312x; 1.1183x over previous
import jax
import jax.numpy as jnp
from jax import lax
from jax.experimental import pallas as pl
from jax.experimental.pallas import tpu as pltpu
from jax.experimental.pallas import tpu_sc as plsc

NC, NS, L = 2, 16, 16
NW = NC * NS
B, H, D = 16384, 200, 10
DP = 16
RPW = B // NW
CH = 16
G = RPW // CH
CI = CH * H
NHID = 5
NUM_ROWS = 1000008
WB1 = NHID * L


def _body(x_hbm, tbl_hbm, wts_hbm, out_hbm,
          idx_v, rows_v, wts_v, out_v, sem, gsem):
    wid = lax.axis_index("s") * NC + lax.axis_index("c")

    pltpu.sync_copy(wts_hbm, wts_v)
    w1vecs = [wts_v[pl.ds(k * L, L)] for k in range(NHID)]
    w1s = [[w1vecs[k][d] for d in range(D)] for k in range(NHID)]
    tail = wts_v[pl.ds(WB1, L)]
    b1s = [tail[k] for k in range(NHID)]
    w2s = [tail[NHID + k] for k in range(NHID)]
    b2s = tail[2 * NHID]

    lane = lax.iota(jnp.int32, L)
    z16 = jnp.zeros((L,), jnp.float32)

    def start(g, buf):
        pltpu.sync_copy(x_hbm.at[wid, g], idx_v.at[buf])
        pltpu.async_copy(tbl_hbm.at[idx_v.at[buf]], rows_v.at[buf], gsem)

    start(0, 0)

    def chunk(g, carry):
        buf = lax.rem(g, 2)
        pltpu.make_async_copy(tbl_hbm.at[idx_v.at[buf]],
                              rows_v.at[buf], gsem).wait()
        pl.when(g + 1 < G)(lambda: start(g + 1, 1 - buf))

        zvec = z16
        for s in range(CH):
            base = s * H

            def inner(u, accs):
                a0, a1, a2, a3 = accs
                r = base + u * 8
                a0 = a0 + rows_v[buf, r]
                a1 = a1 + rows_v[buf, r + 1]
                a2 = a2 + rows_v[buf, r + 2]
                a3 = a3 + rows_v[buf, r + 3]
                a0 = a0 + rows_v[buf, r + 4]
                a1 = a1 + rows_v[buf, r + 5]
                a2 = a2 + rows_v[buf, r + 6]
                a3 = a3 + rows_v[buf, r + 7]
                return a0, a1, a2, a3

            a0, a1, a2, a3 = lax.fori_loop(0, H // 8, inner,
                                           (z16, z16, z16, z16))
            acc = (a0 + a1) + (a2 + a3)

            es = [acc[d] for d in range(D)]
            z = b2s
            for k in range(NHID):
                t = b1s[k]
                for d in range(D):
                    t = t + es[d] * w1s[k][d]
                z = z + w2s[k] * jnp.maximum(t, 0.0)
            zvec = jnp.where(lane == s, z, zvec)

        out_v[pl.ds(g * CH, CH)] = 1.0 / (1.0 + jnp.exp(-zvec))
        return carry

    lax.fori_loop(0, G, chunk, 0)
    pltpu.sync_copy(out_v, out_hbm.at[pl.ds(wid * RPW, RPW)])


def kernel(x, emb_table, w1, b1, w2, b2):
    tbl = jnp.pad(emb_table,
                  ((0, NUM_ROWS - emb_table.shape[0]), (0, DP - D)))
    x_r = x.reshape(NW, G, CI)
    w1p = jnp.pad((w1 / H).astype(jnp.float32), ((0, 0), (0, L - D)))
    wts = jnp.concatenate([
        w1p.reshape(-1),
        b1.reshape(-1),
        w2.reshape(-1),
        b2.reshape(-1),
        jnp.zeros((5,), jnp.float32),
    ])

    run = pl.kernel(
        _body,
        out_type=jax.ShapeDtypeStruct((B,), jnp.float32),
        mesh=plsc.VectorSubcoreMesh(core_axis_name="c", subcore_axis_name="s"),
        compiler_params=pltpu.CompilerParams(use_tc_tiling_on_sc=False),
        scratch_types=[
            pltpu.VMEM((2, CI), jnp.int32),
            pltpu.VMEM((2, CI, DP), jnp.float32),
            pltpu.VMEM((96,), jnp.float32),
            pltpu.VMEM((RPW,), jnp.float32),
            pltpu.SemaphoreType.DMA,
            pltpu.SemaphoreType.DMA,
        ],
    )
    out = run(x_r, tbl, wts)
    return out.reshape(B, 1)
